# baseline (device time: 385582 ns/iter reference)
import jax
import jax.numpy as jnp
from jax import lax
from jax.experimental import pallas as pl
from jax.experimental.pallas import tpu as pltpu

N_DEV = 4
KS = 1024
HS = KS // 2

BM = 1024
BN = 2048


def _ag_body(x_hbm, w_hbm, xg_ref, wg_ref, send_s, recv_s, copy_sem):
    my = lax.axis_index("i")
    left = lax.rem(my + N_DEV - 1, N_DEV)
    right = lax.rem(my + 1, N_DEV)

    cx = pltpu.make_async_copy(x_hbm, xg_ref.at[:, pl.ds(0, KS)], copy_sem)
    cx.start()
    cw = pltpu.make_async_copy(w_hbm, wg_ref.at[pl.ds(0, KS), :], copy_sem)
    cw.start()

    barrier = pltpu.get_barrier_semaphore()
    for nbr in (left, right):
        pl.semaphore_signal(barrier, inc=1, device_id=(nbr,),
                            device_id_type=pl.DeviceIdType.MESH)
    pl.semaphore_wait(barrier, 2)
    cx.wait()
    cw.wait()

    def seg(h, is_left):
        return pl.ds(h * KS + (HS if is_left else 0), HS)

    def flows(h):
        for f, (is_x, is_left) in enumerate(
                [(True, False), (True, True), (False, False), (False, True)]):
            if is_x:
                src = xg_ref.at[:, seg(h, is_left)]
                dst = xg_ref.at[:, seg(h + 1, is_left)]
            else:
                src = wg_ref.at[seg(h, is_left), :]
                dst = wg_ref.at[seg(h + 1, is_left), :]
            yield f, src, dst, is_left

    send_descs = []

    def send_hop(h):
        for f, src, dst, is_left in flows(h):
            d = pltpu.make_async_remote_copy(
                src_ref=src, dst_ref=dst,
                send_sem=send_s.at[f, h], recv_sem=recv_s.at[f, h],
                device_id=(left if is_left else right,),
                device_id_type=pl.DeviceIdType.MESH)
            d.start()
            send_descs.append(d)

    def recv_hop(h):
        for f, src, dst, is_left in flows(h):
            d = pltpu.make_async_remote_copy(
                src_ref=dst, dst_ref=dst,
                send_sem=send_s.at[f, h], recv_sem=recv_s.at[f, h],
                device_id=(right if is_left else left,),
                device_id_type=pl.DeviceIdType.MESH)
            d.wait_recv()

    send_hop(0)
    for h in range(N_DEV - 1):
        recv_hop(h)
        if h < N_DEV - 2:
            send_hop(h + 1)
    for d in send_descs:
        d.wait_send()


def _all_gather(xc, wc):
    m, kx = xc.shape
    kw, n = wc.shape
    dma43 = pltpu.SemaphoreType.DMA((4, N_DEV - 1))
    return pl.pallas_call(
        _ag_body,
        out_shape=[
            jax.ShapeDtypeStruct((m, N_DEV * kx), xc.dtype),
            jax.ShapeDtypeStruct((N_DEV * kw, n), wc.dtype),
        ],
        in_specs=[pl.BlockSpec(memory_space=pl.ANY),
                  pl.BlockSpec(memory_space=pl.ANY)],
        out_specs=[pl.BlockSpec(memory_space=pl.ANY),
                   pl.BlockSpec(memory_space=pl.ANY)],
        scratch_shapes=[dma43, dma43, pltpu.SemaphoreType.DMA],
        compiler_params=pltpu.CompilerParams(collective_id=0),
    )(xc, wc)


def _gemm_body(s_ref, x_ref, w_ref, o_ref):
    o_ref[...] = (
        jnp.dot(x_ref[...], w_ref[...], preferred_element_type=jnp.float32)
        * s_ref[0, 0]
    )


def _gemm(s, xg, wg):
    m, k = xg.shape
    _, n = wg.shape
    return pl.pallas_call(
        _gemm_body,
        grid=(n // BN, m // BM),
        in_specs=[
            pl.BlockSpec((1, 1), lambda j, i: (0, 0),
                         memory_space=pltpu.SMEM),
            pl.BlockSpec((BM, k), lambda j, i: (i, 0)),
            pl.BlockSpec((k, BN), lambda j, i: (0, j)),
        ],
        out_specs=pl.BlockSpec((BM, BN), lambda j, i: (i, j)),
        out_shape=jax.ShapeDtypeStruct((m, n), jnp.float32),
        compiler_params=pltpu.CompilerParams(
            dimension_semantics=("parallel", "parallel"),
            vmem_limit_bytes=56 * 1024 * 1024,
        ),
    )(s, xg, wg)


def kernel(x, w_mat, scale_x, scale_w):
    xc = x.astype(jnp.float8_e4m3fn)
    wc = w_mat.astype(jnp.float8_e5m2)
    xg, wg = _all_gather(xc, wc)
    s = (scale_x * scale_w).reshape(1, 1)
    return _gemm(s, xg, wg)
